# baseline (speedup 1.0000x reference)
import functools

import jax
import jax.numpy as jnp
from jax import lax
from jax.experimental import pallas as pl
from jax.experimental.pallas import tpu as pltpu
from jax.experimental.pallas import tpu_sc as plsc

B = 1024
L = 200
D = 128
MAXLEN = 200
NC = 2
NS = 16
NW = NC * NS
ROWS_PER_W = B // NW
TOK_PER_W = ROWS_PER_W * L
TPAD = 208
CHUNKS = ((0, 128), (128, 72))


def _table_body(pos_ref, mod_ref, out_ref):
    pos = pos_ref[...]
    mod = mod_ref[...]
    pad = jnp.broadcast_to(jnp.zeros_like(mod), (TPAD - MAXLEN, D))
    out_ref[...] = jnp.concatenate([pos, pad], axis=0) + mod


def _build_table(pos200, mod2d):
    return pl.pallas_call(
        _table_body,
        out_shape=jax.ShapeDtypeStruct((TPAD, D), jnp.float32),
    )(pos200, mod2d)


def _sc_body(ids_hbm, mask_hbm, tok_tab, pos_tab, x_hbm, xe_hbm,
             ids_v, mask_v, pid_v, tbuf, pbuf, sem_t, sem_p):
    wid = lax.axis_index("s") * NC + lax.axis_index("c")
    tok0 = wid * TOK_PER_W

    pltpu.sync_copy(ids_hbm.at[pl.ds(tok0, TOK_PER_W)],
                    ids_v.at[pl.ds(0, TOK_PER_W)])
    pltpu.sync_copy(mask_hbm.at[pl.ds(tok0, TOK_PER_W)],
                    mask_v.at[pl.ds(0, TOK_PER_W)])

    lane = lax.iota(jnp.int32, 16)

    def pid_row(r, _):
        rb = r * L
        carry = jnp.int32(-1)
        for i in range(12):
            off = rb + i * 16
            m = mask_v[pl.ds(off, 16)]
            nm = jnp.where(m != 0, 0, 1)
            pid = plsc.cumsum(nm) + carry
            pid_v[pl.ds(off, 16)] = jnp.where(m != 0, MAXLEN, pid)
            carry = carry + jnp.sum(nm)
        off = rb + 192
        m = mask_v[pl.ds(off, 16)]
        nm = jnp.where(jnp.logical_and(lane < 8, m == 0), 1, 0)
        pid = plsc.cumsum(nm) + carry
        pid_v[pl.ds(off, 16)] = jnp.where(m != 0, MAXLEN, pid)
        return 0

    lax.fori_loop(0, ROWS_PER_W, pid_row, 0)

    def gather_row(r, _):
        rb = r * L
        for (o, n) in CHUNKS:
            off = rb + o
            cp_t = pltpu.async_copy(tok_tab.at[ids_v.at[pl.ds(off, n)]],
                                    tbuf.at[pl.ds(0, n)], sem_t)
            cp_p = pltpu.async_copy(pos_tab.at[pid_v.at[pl.ds(off, n)]],
                                    pbuf.at[pl.ds(0, n)], sem_p)
            cp_t.wait()
            pltpu.sync_copy(tbuf.at[pl.ds(0, n)],
                            x_hbm.at[pl.ds(tok0 + off, n)])
            cp_p.wait()
            pltpu.sync_copy(pbuf.at[pl.ds(0, n)],
                            xe_hbm.at[pl.ds(tok0 + off, n)])
        return 0

    lax.fori_loop(0, ROWS_PER_W, gather_row, 0)


_sc_gather = pl.kernel(
    _sc_body,
    out_type=(jax.ShapeDtypeStruct((B * L, D), jnp.float32),
              jax.ShapeDtypeStruct((B * L, D), jnp.float32)),
    mesh=plsc.VectorSubcoreMesh(core_axis_name="c", subcore_axis_name="s"),
    compiler_params=pltpu.CompilerParams(needs_layout_passes=False),
    scratch_types=[
        pltpu.VMEM((TOK_PER_W + 16,), jnp.int32),
        pltpu.VMEM((TOK_PER_W + 16,), jnp.int32),
        pltpu.VMEM((TOK_PER_W + 16,), jnp.int32),
        pltpu.VMEM((128, D), jnp.float32),
        pltpu.VMEM((128, D), jnp.float32),
        pltpu.SemaphoreType.DMA,
        pltpu.SemaphoreType.DMA,
    ],
)


def kernel(tensor, target_mask, token_emb, mod_emb, pos_emb):
    ids = tensor.reshape(B * L)
    mask = target_mask.astype(jnp.int32).reshape(B * L)
    table = _build_table(pos_emb[0, :MAXLEN, :], mod_emb[0])
    x_flat, xe_flat = _sc_gather(ids, mask, token_emb, table)
    return (x_flat.reshape(B, L, D), xe_flat.reshape(B, L, D), tensor)
